# Initial kernel scaffold; baseline (speedup 1.0000x reference)
import jax
import jax.numpy as jnp
from jax.experimental import pallas as pl


def kernel(input, weights):
    raise NotImplementedError("write your pallas kernel here")


# speedup vs baseline: 1.8441x; 1.8441x over previous
import jax
import jax.numpy as jnp
from jax import lax
from jax.experimental import pallas as pl
from jax.experimental.pallas import tpu as pltpu
from jax.experimental.pallas import tpu_sc as plsc

PAD = 1
BSZ = 4
SEQ = 4096
DIM = 1024
N = BSZ * SEQ
NW = 32
CHUNK = N // NW
GC = 64
NCH = CHUNK // GC


def _pos_body(inp_ref, pos_ref):
    x = inp_ref[...]
    m = (x != PAD).astype(jnp.int32)
    c = m
    k = 1
    while k < SEQ:
        z = jnp.zeros((BSZ, k), jnp.int32)
        c = c + jnp.concatenate([z, c[:, : SEQ - k]], axis=1)
        k *= 2
    pos_ref[...] = c * m + 1


def _gather_body(idx_hbm, tab_hbm, out_hbm, idx_v, rows_v, dsem):
    cid = lax.axis_index("c")
    sid = lax.axis_index("s")
    w = sid * 2 + cid
    t0 = pl.multiple_of(w * CHUNK, CHUNK)
    pltpu.sync_copy(idx_hbm.at[pl.ds(t0, CHUNK)], idx_v)
    for g in range(NCH):
        off = g * GC
        pltpu.async_copy(tab_hbm.at[idx_v.at[pl.ds(off, GC)]], rows_v, dsem).wait()
        pltpu.sync_copy(rows_v, out_hbm.at[pl.ds(t0 + off, GC)])


def kernel(input, weights):
    positions = pl.pallas_call(
        _pos_body,
        out_shape=jax.ShapeDtypeStruct((BSZ, SEQ), jnp.int32),
    )(input)

    mesh = plsc.VectorSubcoreMesh(core_axis_name="c", subcore_axis_name="s")
    gather = pl.kernel(
        _gather_body,
        mesh=mesh,
        out_type=jax.ShapeDtypeStruct((N, DIM), jnp.float32),
        scratch_types=[
            pltpu.VMEM((CHUNK,), jnp.int32),
            pltpu.VMEM((GC, DIM), jnp.float32),
            pltpu.SemaphoreType.DMA,
        ],
    )
    out = gather(positions.reshape(-1), weights)
    return out.reshape(BSZ, SEQ, DIM)


# speedup vs baseline: 2.0027x; 1.0860x over previous
import jax
import jax.numpy as jnp
from jax import lax
from jax.experimental import pallas as pl
from jax.experimental.pallas import tpu as pltpu
from jax.experimental.pallas import tpu_sc as plsc

PAD = 1
BSZ = 4
SEQ = 4096
DIM = 1024
N = BSZ * SEQ
NW = 32
CHUNK = N // NW
GC = 32
NCH = CHUNK // GC


def _pos_body(inp_ref, pos_ref):
    x = inp_ref[...]
    m = (x != PAD).astype(jnp.int32)
    c = m
    k = 1
    while k < SEQ:
        z = jnp.zeros((BSZ, k), jnp.int32)
        c = c + jnp.concatenate([z, c[:, : SEQ - k]], axis=1)
        k *= 2
    pos_ref[...] = c * m + 1


def _gather_body(idx_hbm, tab_hbm, out_hbm, idx_v, rows_v, gs0, gs1, ss0, ss1):
    cid = lax.axis_index("c")
    sid = lax.axis_index("s")
    w = sid * 2 + cid
    t0 = pl.multiple_of(w * CHUNK, CHUNK)
    pltpu.sync_copy(idx_hbm.at[pl.ds(t0, CHUNK)], idx_v)

    gsem = [gs0, gs1]
    ssem = [ss0, ss1]

    def start_gather(g, b):
        return pltpu.async_copy(
            tab_hbm.at[idx_v.at[pl.ds(g * GC, GC)]], rows_v.at[b], gsem[b]
        )

    def start_store(g, b):
        return pltpu.async_copy(
            rows_v.at[b], out_hbm.at[pl.ds(t0 + g * GC, GC)], ssem[b]
        )

    gcp = {0: start_gather(0, 0)}
    scp = {}
    for g in range(NCH):
        b = g % 2
        if g + 1 < NCH:
            if g >= 1:
                scp[g - 1].wait()
            gcp[g + 1] = start_gather(g + 1, 1 - b)
        gcp[g].wait()
        scp[g] = start_store(g, b)
    scp[NCH - 2].wait()
    scp[NCH - 1].wait()


def kernel(input, weights):
    positions = pl.pallas_call(
        _pos_body,
        out_shape=jax.ShapeDtypeStruct((BSZ, SEQ), jnp.int32),
    )(input)

    mesh = plsc.VectorSubcoreMesh(core_axis_name="c", subcore_axis_name="s")
    gather = pl.kernel(
        _gather_body,
        mesh=mesh,
        out_type=jax.ShapeDtypeStruct((N, DIM), jnp.float32),
        scratch_types=[
            pltpu.VMEM((CHUNK,), jnp.int32),
            pltpu.VMEM((2, GC, DIM), jnp.float32),
            pltpu.SemaphoreType.DMA,
            pltpu.SemaphoreType.DMA,
            pltpu.SemaphoreType.DMA,
            pltpu.SemaphoreType.DMA,
        ],
    )
    out = gather(positions.reshape(-1), weights)
    return out.reshape(BSZ, SEQ, DIM)
